# baseline (speedup 1.0000x reference)
import functools

import jax
import jax.numpy as jnp
from jax import lax
from jax.experimental import pallas as pl
from jax.experimental.pallas import tpu as pltpu
from jax.experimental.pallas import tpu_sc as plsc

_NUM_ROWS = 424
_PAIR_DIM = 128
_LAG_DIM = 16
_OUT_DIM = _PAIR_DIM + _LAG_DIM
_NUM_LAGS = 4
_CHUNK = 16
_ACTIVE = (_NUM_ROWS + _CHUNK - 1) // _CHUNK
_NUM_CORES = 2


def _take(v, i):
    dnums = lax.GatherDimensionNumbers(
        offset_dims=(), collapsed_slice_dims=(0,), start_index_map=(0,))
    return lax.gather(v, i[:, None], dnums, slice_sizes=(1,),
                      mode=lax.GatherScatterMode.PROMISE_IN_BOUNDS)


def _body(idx_hbm, pair_hbm, lag_hbm, out_hbm,
          idx_v, idxp_v, idxl_v, pair_v, lag_v, sem_p, sem_l):
    wid = lax.axis_index("s") * _NUM_CORES + lax.axis_index("c")

    @pl.when(wid < _ACTIVE)
    def _():
        base = lax.min(wid * _CHUNK, _NUM_ROWS - _CHUNK)
        pltpu.sync_copy(idx_hbm.at[pl.ds(2 * base, 2 * _CHUNK)], idx_v)

        iota = lax.iota(jnp.int32, 16)
        va = idx_v[pl.ds(0, 16)]
        vb = idx_v[pl.ds(16, 16)]
        geven = (2 * iota) % 16
        half = iota < 8
        idxp_v[...] = jnp.where(half, _take(va, geven), _take(vb, geven))
        idxl_v[...] = jnp.where(half, _take(va, geven + 1),
                                _take(vb, geven + 1))

        cp = pltpu.async_copy(pair_hbm.at[idxp_v], pair_v, sem_p)
        cl = pltpu.async_copy(lag_hbm.at[idxl_v], lag_v, sem_l)
        cp.wait()
        pltpu.sync_copy(pair_v, out_hbm.at[pl.ds(base, _CHUNK),
                                           pl.ds(0, _PAIR_DIM)])
        cl.wait()
        pltpu.sync_copy(lag_v.at[:, pl.ds(0, _LAG_DIM)],
                        out_hbm.at[pl.ds(base, _CHUNK),
                                   pl.ds(_PAIR_DIM, _LAG_DIM)])


@jax.jit
def _emb(idx_flat, pair_table, lag_pad):
    mesh = plsc.VectorSubcoreMesh(core_axis_name="c", subcore_axis_name="s")
    run = functools.partial(
        pl.kernel,
        out_type=jax.ShapeDtypeStruct((_NUM_ROWS, _OUT_DIM), jnp.float32),
        mesh=mesh,
        compiler_params=pltpu.CompilerParams(use_tc_tiling_on_sc=False),
        scratch_types=[
            pltpu.VMEM((2 * _CHUNK,), jnp.int32),
            pltpu.VMEM((_CHUNK,), jnp.int32),
            pltpu.VMEM((_CHUNK,), jnp.int32),
            pltpu.VMEM((_CHUNK, _PAIR_DIM), jnp.float32),
            pltpu.VMEM((_CHUNK, _PAIR_DIM), jnp.float32),
            pltpu.SemaphoreType.DMA,
            pltpu.SemaphoreType.DMA,
        ],
    )(_body)
    return run(idx_flat, pair_table, lag_pad)


def kernel(target_indices, pair_table, lag_table):
    idx_flat = target_indices.astype(jnp.int32).reshape(-1)
    lag_pad = jnp.pad(lag_table, ((0, 0), (0, _PAIR_DIM - _LAG_DIM)))
    return _emb(idx_flat, pair_table, lag_pad)


# speedup vs baseline: 1.1441x; 1.1441x over previous
import functools

import jax
import jax.numpy as jnp
from jax import lax
from jax.experimental import pallas as pl
from jax.experimental.pallas import tpu as pltpu
from jax.experimental.pallas import tpu_sc as plsc

_NUM_ROWS = 424
_PAIR_DIM = 128
_LAG_DIM = 16
_OUT_DIM = _PAIR_DIM + _LAG_DIM
_NUM_LAGS = 4
_CHUNK = 16
_ACTIVE = (_NUM_ROWS + _CHUNK - 1) // _CHUNK
_NUM_CORES = 2


def _take(v, i):
    dnums = lax.GatherDimensionNumbers(
        offset_dims=(), collapsed_slice_dims=(0,), start_index_map=(0,))
    return lax.gather(v, i[:, None], dnums, slice_sizes=(1,),
                      mode=lax.GatherScatterMode.PROMISE_IN_BOUNDS)


def _body(idx_hbm, pair_hbm, lag_hbm, out_hbm,
          idx_v, idxp_v, lag_t_v, pair_v, out_v, sem_i, sem_t, sem_p):
    wid = lax.axis_index("s") * _NUM_CORES + lax.axis_index("c")

    @pl.when(wid < _ACTIVE)
    def _():
        base = lax.min(wid * _CHUNK, _NUM_ROWS - _CHUNK)
        ci = pltpu.async_copy(idx_hbm.at[pl.ds(2 * base, 2 * _CHUNK)],
                              idx_v, sem_i)
        ct = pltpu.async_copy(lag_hbm, lag_t_v, sem_t)
        ci.wait()

        iota = lax.iota(jnp.int32, 16)
        va = idx_v[pl.ds(0, 16)]
        vb = idx_v[pl.ds(16, 16)]
        geven = (2 * iota) % 16
        half = iota < 8
        pidx = jnp.where(half, _take(va, geven), _take(vb, geven))
        lidx = jnp.where(half, _take(va, geven + 1), _take(vb, geven + 1))
        idxp_v[...] = pidx

        cp = pltpu.async_copy(pair_hbm.at[idxp_v], pair_v, sem_p)

        ct.wait()
        lr = [lag_t_v[i, :] for i in range(_NUM_LAGS)]
        d01 = lr[1] - lr[0]
        d23 = lr[3] - lr[2]
        one = jnp.ones((16,), jnp.int32)
        for r in range(_CHUNK):
            bc = _take(lidx, jnp.full((16,), r, jnp.int32))
            b0 = (bc & one).astype(jnp.float32)
            b1 = ((bc >> 1) & one).astype(jnp.float32)
            lo = lr[0] + b0 * d01
            hi = lr[2] + b0 * d23
            sel = lo + b1 * (hi - lo)
            out_v[r, pl.ds(_PAIR_DIM, _LAG_DIM)] = sel

        cp.wait()
        for r in range(_CHUNK):
            for j in range(_PAIR_DIM // 16):
                out_v[r, pl.ds(j * 16, 16)] = pair_v[r, pl.ds(j * 16, 16)]

        pltpu.sync_copy(out_v, out_hbm.at[pl.ds(base, _CHUNK)])


@jax.jit
def _emb(idx_flat, pair_table, lag_table):
    mesh = plsc.VectorSubcoreMesh(core_axis_name="c", subcore_axis_name="s")
    run = functools.partial(
        pl.kernel,
        out_type=jax.ShapeDtypeStruct((_NUM_ROWS, _OUT_DIM), jnp.float32),
        mesh=mesh,
        scratch_types=[
            pltpu.VMEM((2 * _CHUNK,), jnp.int32),
            pltpu.VMEM((_CHUNK,), jnp.int32),
            pltpu.VMEM((_NUM_LAGS, _LAG_DIM), jnp.float32),
            pltpu.VMEM((_CHUNK, _PAIR_DIM), jnp.float32),
            pltpu.VMEM((_CHUNK, _OUT_DIM), jnp.float32),
            pltpu.SemaphoreType.DMA,
            pltpu.SemaphoreType.DMA,
            pltpu.SemaphoreType.DMA,
        ],
    )(_body)
    return run(idx_flat, pair_table, lag_table)


def kernel(target_indices, pair_table, lag_table):
    idx_flat = target_indices.astype(jnp.int32).reshape(-1)
    return _emb(idx_flat, pair_table, lag_table)


# speedup vs baseline: 1.1487x; 1.0040x over previous
import functools

import jax
import jax.numpy as jnp
from jax import lax
from jax.experimental import pallas as pl
from jax.experimental.pallas import tpu as pltpu
from jax.experimental.pallas import tpu_sc as plsc

_NUM_ROWS = 424
_PAIR_DIM = 128
_LAG_DIM = 16
_OUT_DIM = _PAIR_DIM + _LAG_DIM
_NUM_LAGS = 4
_CHUNK = 16
_ACTIVE = (_NUM_ROWS + _CHUNK - 1) // _CHUNK
_NUM_CORES = 2


def _take(v, i):
    dnums = lax.GatherDimensionNumbers(
        offset_dims=(), collapsed_slice_dims=(0,), start_index_map=(0,))
    return lax.gather(v, i[:, None], dnums, slice_sizes=(1,),
                      mode=lax.GatherScatterMode.PROMISE_IN_BOUNDS)


def _body(idx_hbm, pair_hbm, lag_hbm, out_hbm,
          idx_v, idxp_v, lag_t_v, pair_v, out_v, sem_i, sem_t, sem_p):
    wid = lax.axis_index("s") * _NUM_CORES + lax.axis_index("c")

    @pl.when(wid < _ACTIVE)
    def _():
        base = lax.min(wid * _CHUNK, _NUM_ROWS - _CHUNK)
        ci = pltpu.async_copy(idx_hbm.at[pl.ds(2 * base, 2 * _CHUNK)],
                              idx_v, sem_i)
        ct = pltpu.async_copy(lag_hbm, lag_t_v, sem_t)
        ci.wait()

        iota = lax.iota(jnp.int32, 16)
        va = idx_v[pl.ds(0, 16)]
        vb = idx_v[pl.ds(16, 16)]
        geven = (2 * iota) % 16
        half = iota < 8
        pidx = jnp.where(half, _take(va, geven), _take(vb, geven))
        lidx = jnp.where(half, _take(va, geven + 1), _take(vb, geven + 1))
        idxp_v[...] = pidx

        cp = pltpu.async_copy(pair_hbm.at[idxp_v], pair_v, sem_p)

        ct.wait()
        lr = [lag_t_v[i, :] for i in range(_NUM_LAGS)]
        d01 = lr[1] - lr[0]
        d23 = lr[3] - lr[2]
        one = jnp.ones((16,), jnp.int32)
        for r in range(_CHUNK):
            bc = _take(lidx, jnp.full((16,), r, jnp.int32))
            b0 = (bc & one).astype(jnp.float32)
            b1 = ((bc >> 1) & one).astype(jnp.float32)
            lo = lr[0] + b0 * d01
            hi = lr[2] + b0 * d23
            sel = lo + b1 * (hi - lo)
            out_v[r, pl.ds(_PAIR_DIM, _LAG_DIM)] = sel

        cp.wait()
        for r in range(_CHUNK):
            for j in range(_PAIR_DIM // 16):
                out_v[r, pl.ds(j * 16, 16)] = pair_v[r, pl.ds(j * 16, 16)]

        pltpu.sync_copy(out_v, out_hbm.at[pl.ds(base, _CHUNK)])


@jax.jit
def _emb(idx_flat, pair_table, lag_table):
    mesh = plsc.VectorSubcoreMesh(core_axis_name="c", subcore_axis_name="s")
    run = functools.partial(
        pl.kernel,
        out_type=jax.ShapeDtypeStruct((_NUM_ROWS, _OUT_DIM), jnp.float32),
        mesh=mesh,
        compiler_params=pltpu.CompilerParams(skip_device_barrier=True),
        scratch_types=[
            pltpu.VMEM((2 * _CHUNK,), jnp.int32),
            pltpu.VMEM((_CHUNK,), jnp.int32),
            pltpu.VMEM((_NUM_LAGS, _LAG_DIM), jnp.float32),
            pltpu.VMEM((_CHUNK, _PAIR_DIM), jnp.float32),
            pltpu.VMEM((_CHUNK, _OUT_DIM), jnp.float32),
            pltpu.SemaphoreType.DMA,
            pltpu.SemaphoreType.DMA,
            pltpu.SemaphoreType.DMA,
        ],
    )(_body)
    return run(idx_flat, pair_table, lag_table)


def kernel(target_indices, pair_table, lag_table):
    idx_flat = target_indices.astype(jnp.int32).reshape(-1)
    return _emb(idx_flat, pair_table, lag_table)


# speedup vs baseline: 1.2209x; 1.0629x over previous
import functools

import jax
import jax.numpy as jnp
from jax import lax
from jax.experimental import pallas as pl
from jax.experimental.pallas import tpu as pltpu
from jax.experimental.pallas import tpu_sc as plsc

_NUM_ROWS = 424
_PAIR_DIM = 128
_LAG_DIM = 16
_OUT_DIM = _PAIR_DIM + _LAG_DIM
_NUM_LAGS = 4
_CHUNK = 32
_GROUPS = _CHUNK // 16
_ACTIVE = (_NUM_ROWS + _CHUNK - 1) // _CHUNK
_NUM_CORES = 1


def _take(v, i):
    dnums = lax.GatherDimensionNumbers(
        offset_dims=(), collapsed_slice_dims=(0,), start_index_map=(0,))
    return lax.gather(v, i[:, None], dnums, slice_sizes=(1,),
                      mode=lax.GatherScatterMode.PROMISE_IN_BOUNDS)


def _body(idx_hbm, pair_hbm, lag_hbm, out_hbm,
          idx_v, idxp_v, lag_t_v, pair_v, out_v, sem_i, sem_t, sem_p):
    wid = lax.axis_index("s") * _NUM_CORES + lax.axis_index("c")

    @pl.when(wid < _ACTIVE)
    def _():
        base = lax.min(wid * _CHUNK, _NUM_ROWS - _CHUNK)
        ci = pltpu.async_copy(idx_hbm.at[pl.ds(2 * base, 2 * _CHUNK)],
                              idx_v, sem_i)
        ct = pltpu.async_copy(lag_hbm, lag_t_v, sem_t)
        ci.wait()

        iota = lax.iota(jnp.int32, 16)
        geven = (2 * iota) % 16
        half = iota < 8
        lidx = []
        for g in range(_GROUPS):
            va = idx_v[pl.ds(32 * g, 16)]
            vb = idx_v[pl.ds(32 * g + 16, 16)]
            pidx = jnp.where(half, _take(va, geven), _take(vb, geven))
            lidx.append(jnp.where(half, _take(va, geven + 1),
                                  _take(vb, geven + 1)))
            idxp_v[pl.ds(16 * g, 16)] = pidx

        cp = pltpu.async_copy(pair_hbm.at[idxp_v], pair_v, sem_p)

        ct.wait()
        lr = [lag_t_v[i, :] for i in range(_NUM_LAGS)]
        d01 = lr[1] - lr[0]
        d23 = lr[3] - lr[2]
        one = jnp.ones((16,), jnp.int32)
        for r in range(_CHUNK):
            bc = _take(lidx[r // 16], jnp.full((16,), r % 16, jnp.int32))
            b0 = (bc & one).astype(jnp.float32)
            b1 = ((bc >> 1) & one).astype(jnp.float32)
            lo = lr[0] + b0 * d01
            hi = lr[2] + b0 * d23
            sel = lo + b1 * (hi - lo)
            out_v[r, pl.ds(_PAIR_DIM, _LAG_DIM)] = sel

        cp.wait()
        for r in range(_CHUNK):
            for j in range(_PAIR_DIM // 16):
                out_v[r, pl.ds(j * 16, 16)] = pair_v[r, pl.ds(j * 16, 16)]

        pltpu.sync_copy(out_v, out_hbm.at[pl.ds(base, _CHUNK)])


@jax.jit
def _emb(idx_flat, pair_table, lag_table):
    mesh = plsc.VectorSubcoreMesh(core_axis_name="c", subcore_axis_name="s",
                                  num_cores=_NUM_CORES)
    run = functools.partial(
        pl.kernel,
        out_type=jax.ShapeDtypeStruct((_NUM_ROWS, _OUT_DIM), jnp.float32),
        mesh=mesh,
        compiler_params=pltpu.CompilerParams(skip_device_barrier=True),
        scratch_types=[
            pltpu.VMEM((2 * _CHUNK,), jnp.int32),
            pltpu.VMEM((_CHUNK,), jnp.int32),
            pltpu.VMEM((_NUM_LAGS, _LAG_DIM), jnp.float32),
            pltpu.VMEM((_CHUNK, _PAIR_DIM), jnp.float32),
            pltpu.VMEM((_CHUNK, _OUT_DIM), jnp.float32),
            pltpu.SemaphoreType.DMA,
            pltpu.SemaphoreType.DMA,
            pltpu.SemaphoreType.DMA,
        ],
    )(_body)
    return run(idx_flat, pair_table, lag_table)


def kernel(target_indices, pair_table, lag_table):
    idx_flat = target_indices.astype(jnp.int32).reshape(-1)
    return _emb(idx_flat, pair_table, lag_table)
